# Initial kernel scaffold; baseline (speedup 1.0000x reference)
import jax
import jax.numpy as jnp
from jax.experimental import pallas as pl


def kernel(x, edge_index, edge_attr, W_rel1, b_rel1, W_root1, W_rel2, b_rel2, W_root2):
    raise NotImplementedError("write your pallas kernel here")


# speedup vs baseline: 10.8782x; 10.8782x over previous
import functools

import jax
import jax.numpy as jnp
from jax import lax
from jax.experimental import pallas as pl
from jax.experimental.pallas import tpu as pltpu
from jax.experimental.pallas import tpu_sc as plsc

N = 10000
E = 320000
D_IN = 128
D_H = 16
D_OUT = 128

NC = 2
NS = 16
NW = NC * NS
GRP = 128
GPW = 80
E_PAD = NW * GPW * GRP
N_PAD = 10240
RPT = N_PAD // NS

_mesh = plsc.VectorSubcoreMesh(
    core_axis_name="c", subcore_axis_name="s", num_cores=NC, num_subcores=NS
)


@functools.partial(
    pl.kernel,
    mesh=_mesh,
    compiler_params=pltpu.CompilerParams(use_tc_tiling_on_sc=False),
    out_type=jax.ShapeDtypeStruct((NC, N_PAD, D_H), jnp.float32),
    scratch_types=[
        pltpu.VMEM((GPW, GRP), jnp.int32),
        pltpu.VMEM((GPW, GRP), jnp.int32),
        pltpu.VMEM((GPW, GRP), jnp.float32),
        pltpu.VMEM((GRP, D_H), jnp.float32),
        pltpu.VMEM((GRP, D_H), jnp.float32),
        pltpu.VMEM_SHARED((N_PAD, D_H), jnp.float32),
        pltpu.SemaphoreType.DMA,
    ],
)
def _segsum_sc(table_hbm, src_hbm, dst_hbm, w_hbm, out_hbm,
               src_v, dst_v, w_v, rows_v, zero_v, acc_sh, sem):
    c = lax.axis_index("c")
    s = lax.axis_index("s")
    wid = c * NS + s

    zrow = jnp.zeros((D_H,), jnp.float32)

    def _zfill(i, carry):
        zero_v[i, :] = zrow
        return carry

    lax.fori_loop(0, GRP, _zfill, 0)

    def _zcopy(t, carry):
        pltpu.sync_copy(
            zero_v,
            acc_sh.at[pl.ds(s * RPT + t * GRP, GRP)],
        )
        return carry

    lax.fori_loop(0, RPT // GRP, _zcopy, 0)
    plsc.subcore_barrier()

    pltpu.sync_copy(src_hbm.at[pl.ds(wid * GPW, GPW)], src_v)
    pltpu.sync_copy(dst_hbm.at[pl.ds(wid * GPW, GPW)], dst_v)
    pltpu.sync_copy(w_hbm.at[pl.ds(wid * GPW, GPW)], w_v)

    def _group(g, carry):
        pltpu.async_copy(table_hbm.at[src_v.at[g]], rows_v, sem).wait()

        def _scale(j, carry2):
            w16 = w_v[g, pl.ds(j * D_H, D_H)]
            base = j * D_H
            for k in range(D_H):
                rows_v[base + k, :] = rows_v[base + k, :] * w16[k]
            return carry2

        lax.fori_loop(0, GRP // D_H, _scale, 0)
        pltpu.sync_copy(rows_v, acc_sh.at[dst_v.at[g]], add=True)
        return carry

    lax.fori_loop(0, GPW, _group, 0)
    plsc.subcore_barrier()

    pltpu.sync_copy(
        acc_sh.at[pl.ds(s * RPT, RPT)],
        out_hbm.at[c, pl.ds(s * RPT, RPT)],
    )


_BN = 2000


def _p1_body(x_ref, wr_ref, wo_ref, xr_ref, xo_ref):
    x = x_ref[...]
    xr_ref[...] = jnp.dot(x, wr_ref[...], preferred_element_type=jnp.float32)
    xo_ref[...] = jnp.dot(x, wo_ref[...], preferred_element_type=jnp.float32)


def _p1(x, W_rel1, W_root1):
    return pl.pallas_call(
        _p1_body,
        grid=(N // _BN,),
        in_specs=[
            pl.BlockSpec((_BN, D_IN), lambda i: (i, 0)),
            pl.BlockSpec((D_IN, D_H), lambda i: (0, 0)),
            pl.BlockSpec((D_IN, D_H), lambda i: (0, 0)),
        ],
        out_specs=[
            pl.BlockSpec((_BN, D_H), lambda i: (i, 0)),
            pl.BlockSpec((_BN, D_H), lambda i: (i, 0)),
        ],
        out_shape=[
            jax.ShapeDtypeStruct((N, D_H), jnp.float32),
            jax.ShapeDtypeStruct((N, D_H), jnp.float32),
        ],
    )(x, W_rel1, W_root1)


def _p2_body(p0_ref, p1_ref, xo_ref, b_ref, h_ref):
    h = p0_ref[...] + p1_ref[...] + xo_ref[...] + b_ref[...]
    h_ref[...] = jnp.maximum(h, 0.0)


def _p2(p0, p1, xo, b1):
    return pl.pallas_call(
        _p2_body,
        grid=(N // _BN,),
        in_specs=[
            pl.BlockSpec((_BN, D_H), lambda i: (i, 0)),
            pl.BlockSpec((_BN, D_H), lambda i: (i, 0)),
            pl.BlockSpec((_BN, D_H), lambda i: (i, 0)),
            pl.BlockSpec((1, D_H), lambda i: (0, 0)),
        ],
        out_specs=pl.BlockSpec((_BN, D_H), lambda i: (i, 0)),
        out_shape=jax.ShapeDtypeStruct((N, D_H), jnp.float32),
    )(p0, p1, xo, b1)


def _p3_body(q0_ref, q1_ref, h_ref, wr_ref, wo_ref, b_ref, out_ref):
    agg = q0_ref[...] + q1_ref[...]
    out_ref[...] = (
        jnp.dot(agg, wr_ref[...], preferred_element_type=jnp.float32)
        + jnp.dot(h_ref[...], wo_ref[...], preferred_element_type=jnp.float32)
        + b_ref[...]
    )


def _p3(q0, q1, h, W_rel2, W_root2, b2):
    return pl.pallas_call(
        _p3_body,
        grid=(N // _BN,),
        in_specs=[
            pl.BlockSpec((_BN, D_H), lambda i: (i, 0)),
            pl.BlockSpec((_BN, D_H), lambda i: (i, 0)),
            pl.BlockSpec((_BN, D_H), lambda i: (i, 0)),
            pl.BlockSpec((D_H, D_OUT), lambda i: (0, 0)),
            pl.BlockSpec((D_H, D_OUT), lambda i: (0, 0)),
            pl.BlockSpec((1, D_OUT), lambda i: (0, 0)),
        ],
        out_specs=pl.BlockSpec((_BN, D_OUT), lambda i: (i, 0)),
        out_shape=jax.ShapeDtypeStruct((N, D_OUT), jnp.float32),
    )(q0, q1, h, W_rel2, W_root2, b2)


def kernel(x, edge_index, edge_attr, W_rel1, b_rel1, W_root1,
           W_rel2, b_rel2, W_root2):
    src = edge_index[0]
    dst = edge_index[1]

    pad = E_PAD - E
    src_p = jnp.concatenate([src, jnp.zeros((pad,), jnp.int32)]).reshape(NW * GPW, GRP)
    dst_p = jnp.concatenate([dst, jnp.zeros((pad,), jnp.int32)]).reshape(NW * GPW, GRP)
    w_p = jnp.concatenate([edge_attr, jnp.zeros((pad,), jnp.float32)]).reshape(NW * GPW, GRP)

    xr, xo = _p1(x, W_rel1, W_root1)
    p = _segsum_sc(xr, src_p, dst_p, w_p)
    h = _p2(p[0], p[1], xo, b_rel1.reshape(1, D_H))
    q = _segsum_sc(h, src_p, dst_p, w_p)
    return _p3(q[0], q[1], h, W_rel2, W_root2, b_rel2.reshape(1, D_OUT))


# speedup vs baseline: 15.9612x; 1.4673x over previous
import functools

import jax
import jax.numpy as jnp
from jax import lax
from jax.experimental import pallas as pl
from jax.experimental.pallas import tpu as pltpu
from jax.experimental.pallas import tpu_sc as plsc

N = 10000
E = 320000
D_IN = 128
D_H = 16
D_OUT = 128

NC = 2
NS = 16
NW = NC * NS
GRP = 128
GPW = 80
E_PAD = NW * GPW * GRP
N_PAD = 10240
RPT = N_PAD // NS
NBUF = 4

_mesh = plsc.VectorSubcoreMesh(
    core_axis_name="c", subcore_axis_name="s", num_cores=NC, num_subcores=NS
)


@functools.partial(
    pl.kernel,
    mesh=_mesh,
    compiler_params=pltpu.CompilerParams(use_tc_tiling_on_sc=False),
    out_type=jax.ShapeDtypeStruct((NC, N_PAD, D_H), jnp.float32),
    scratch_types=[
        pltpu.VMEM((GPW, GRP), jnp.int32),
        pltpu.VMEM((GPW, GRP), jnp.int32),
        pltpu.VMEM((GPW, GRP), jnp.float32),
        pltpu.VMEM((NBUF, GRP, D_H), jnp.float32),
        pltpu.VMEM((GRP, D_H), jnp.float32),
        pltpu.VMEM_SHARED((N_PAD, D_H), jnp.float32),
        pltpu.SemaphoreType.DMA,
        [pltpu.SemaphoreType.DMA] * NBUF,
    ],
)
def _segsum_sc(table_hbm, src_hbm, dst_hbm, w_hbm, out_hbm,
               src_v, dst_v, w_v, rows_v, zero_v, acc_sh, stage_sem, gsems):
    c = lax.axis_index("c")
    s = lax.axis_index("s")
    wid = c * NS + s

    stage_src = pltpu.async_copy(src_hbm.at[pl.ds(wid * GPW, GPW)], src_v, stage_sem)
    stage_dst = pltpu.async_copy(dst_hbm.at[pl.ds(wid * GPW, GPW)], dst_v, stage_sem)
    stage_w = pltpu.async_copy(w_hbm.at[pl.ds(wid * GPW, GPW)], w_v, stage_sem)

    zrow = jnp.zeros((D_H,), jnp.float32)

    def _zfill(i, carry):
        zero_v[i, :] = zrow
        return carry

    lax.fori_loop(0, GRP, _zfill, 0)

    def _zcopy(t, carry):
        pltpu.sync_copy(
            zero_v,
            acc_sh.at[pl.ds(s * RPT + t * GRP, GRP)],
        )
        return carry

    lax.fori_loop(0, RPT // GRP, _zcopy, 0)

    stage_src.wait()
    stage_dst.wait()
    stage_w.wait()

    for b in range(NBUF):
        pltpu.async_copy(table_hbm.at[src_v.at[b]], rows_v.at[b], gsems[b])

    plsc.subcore_barrier()

    def _outer(gg, carry):
        for b in range(NBUF):
            g = gg * NBUF + b
            pltpu.make_async_copy(
                table_hbm.at[src_v.at[g]], rows_v.at[b], gsems[b]
            ).wait()

            def _scale(j, carry2, g=g, b=b):
                w16 = w_v[g, pl.ds(j * D_H, D_H)]
                for k in range(D_H):
                    rows_v[b, j * D_H + k, :] = rows_v[b, j * D_H + k, :] * w16[k]
                return carry2

            lax.fori_loop(0, GRP // D_H, _scale, 0)
            pltpu.sync_copy(rows_v.at[b], acc_sh.at[dst_v.at[g]], add=True)

            @pl.when(g + NBUF < GPW)
            def _(g=g, b=b):
                pltpu.async_copy(
                    table_hbm.at[src_v.at[g + NBUF]], rows_v.at[b], gsems[b]
                )

        return carry

    lax.fori_loop(0, GPW // NBUF, _outer, 0)
    plsc.subcore_barrier()

    pltpu.sync_copy(
        acc_sh.at[pl.ds(s * RPT, RPT)],
        out_hbm.at[c, pl.ds(s * RPT, RPT)],
    )


_BN = 2000


def _p1_body(x_ref, wr_ref, wo_ref, xr_ref, xo_ref):
    x = x_ref[...]
    xr_ref[...] = jnp.dot(x, wr_ref[...], preferred_element_type=jnp.float32)
    xo_ref[...] = jnp.dot(x, wo_ref[...], preferred_element_type=jnp.float32)


def _p1(x, W_rel1, W_root1):
    return pl.pallas_call(
        _p1_body,
        grid=(N // _BN,),
        in_specs=[
            pl.BlockSpec((_BN, D_IN), lambda i: (i, 0)),
            pl.BlockSpec((D_IN, D_H), lambda i: (0, 0)),
            pl.BlockSpec((D_IN, D_H), lambda i: (0, 0)),
        ],
        out_specs=[
            pl.BlockSpec((_BN, D_H), lambda i: (i, 0)),
            pl.BlockSpec((_BN, D_H), lambda i: (i, 0)),
        ],
        out_shape=[
            jax.ShapeDtypeStruct((N, D_H), jnp.float32),
            jax.ShapeDtypeStruct((N, D_H), jnp.float32),
        ],
    )(x, W_rel1, W_root1)


def _p2_body(p0_ref, p1_ref, xo_ref, b_ref, h_ref):
    h = p0_ref[...] + p1_ref[...] + xo_ref[...] + b_ref[...]
    h_ref[...] = jnp.maximum(h, 0.0)


def _p2(p0, p1, xo, b1):
    return pl.pallas_call(
        _p2_body,
        grid=(N // _BN,),
        in_specs=[
            pl.BlockSpec((_BN, D_H), lambda i: (i, 0)),
            pl.BlockSpec((_BN, D_H), lambda i: (i, 0)),
            pl.BlockSpec((_BN, D_H), lambda i: (i, 0)),
            pl.BlockSpec((1, D_H), lambda i: (0, 0)),
        ],
        out_specs=pl.BlockSpec((_BN, D_H), lambda i: (i, 0)),
        out_shape=jax.ShapeDtypeStruct((N, D_H), jnp.float32),
    )(p0, p1, xo, b1)


def _p3_body(q0_ref, q1_ref, h_ref, wr_ref, wo_ref, b_ref, out_ref):
    agg = q0_ref[...] + q1_ref[...]
    out_ref[...] = (
        jnp.dot(agg, wr_ref[...], preferred_element_type=jnp.float32)
        + jnp.dot(h_ref[...], wo_ref[...], preferred_element_type=jnp.float32)
        + b_ref[...]
    )


def _p3(q0, q1, h, W_rel2, W_root2, b2):
    return pl.pallas_call(
        _p3_body,
        grid=(N // _BN,),
        in_specs=[
            pl.BlockSpec((_BN, D_H), lambda i: (i, 0)),
            pl.BlockSpec((_BN, D_H), lambda i: (i, 0)),
            pl.BlockSpec((_BN, D_H), lambda i: (i, 0)),
            pl.BlockSpec((D_H, D_OUT), lambda i: (0, 0)),
            pl.BlockSpec((D_H, D_OUT), lambda i: (0, 0)),
            pl.BlockSpec((1, D_OUT), lambda i: (0, 0)),
        ],
        out_specs=pl.BlockSpec((_BN, D_OUT), lambda i: (i, 0)),
        out_shape=jax.ShapeDtypeStruct((N, D_OUT), jnp.float32),
    )(q0, q1, h, W_rel2, W_root2, b2)


def kernel(x, edge_index, edge_attr, W_rel1, b_rel1, W_root1,
           W_rel2, b_rel2, W_root2):
    src = edge_index[0]
    dst = edge_index[1]

    pad = E_PAD - E
    src_p = jnp.concatenate([src, jnp.zeros((pad,), jnp.int32)]).reshape(NW * GPW, GRP)
    dst_p = jnp.concatenate([dst, jnp.zeros((pad,), jnp.int32)]).reshape(NW * GPW, GRP)
    w_p = jnp.concatenate([edge_attr, jnp.zeros((pad,), jnp.float32)]).reshape(NW * GPW, GRP)

    xr, xo = _p1(x, W_rel1, W_root1)
    p = _segsum_sc(xr, src_p, dst_p, w_p)
    h = _p2(p[0], p[1], xo, b_rel1.reshape(1, D_H))
    q = _segsum_sc(h, src_p, dst_p, w_p)
    return _p3(q[0], q[1], h, W_rel2, W_root2, b_rel2.reshape(1, D_OUT))


# speedup vs baseline: 23.1511x; 1.4505x over previous
import functools

import jax
import jax.numpy as jnp
from jax import lax
from jax.experimental import pallas as pl
from jax.experimental.pallas import tpu as pltpu
from jax.experimental.pallas import tpu_sc as plsc

N = 10000
E = 320000
D_IN = 128
D_H = 16
D_OUT = 128

NC = 2
NS = 16
NW = NC * NS
GRP = 128
GPW = 80
E_PAD = NW * GPW * GRP
N_PAD = 10240
RPT = N_PAD // NS
NBUF = 4

_mesh = plsc.VectorSubcoreMesh(
    core_axis_name="c", subcore_axis_name="s", num_cores=NC, num_subcores=NS
)


@functools.partial(
    pl.kernel,
    mesh=_mesh,
    compiler_params=pltpu.CompilerParams(use_tc_tiling_on_sc=False),
    out_type=jax.ShapeDtypeStruct((NC, N_PAD, D_H), jnp.float32),
    scratch_types=[
        pltpu.VMEM((GPW, GRP), jnp.int32),
        pltpu.VMEM((GPW, GRP), jnp.int32),
        pltpu.VMEM((GPW, GRP), jnp.float32),
        pltpu.VMEM((NBUF, GRP, D_H), jnp.float32),
        pltpu.VMEM((GRP, D_H), jnp.float32),
        pltpu.VMEM_SHARED((N_PAD, D_H), jnp.float32),
        pltpu.VMEM_SHARED((N_PAD, D_H), jnp.float32),
        pltpu.SemaphoreType.DMA,
        [pltpu.SemaphoreType.DMA] * NBUF,
    ],
)
def _segsum_sc(table_hbm, src_hbm, dst_hbm, w_hbm, out_hbm,
               src_v, dst_v, w_v, rows_v, zero_v, acc_sh, table_sh, stage_sem, gsems):
    c = lax.axis_index("c")
    s = lax.axis_index("s")
    wid = c * NS + s

    stage_src = pltpu.async_copy(src_hbm.at[pl.ds(wid * GPW, GPW)], src_v, stage_sem)
    stage_dst = pltpu.async_copy(dst_hbm.at[pl.ds(wid * GPW, GPW)], dst_v, stage_sem)
    stage_w = pltpu.async_copy(w_hbm.at[pl.ds(wid * GPW, GPW)], w_v, stage_sem)
    stage_tab = pltpu.async_copy(
        table_hbm.at[pl.ds(s * RPT, RPT)], table_sh.at[pl.ds(s * RPT, RPT)], stage_sem
    )

    zrow = jnp.zeros((D_H,), jnp.float32)

    def _zfill(i, carry):
        zero_v[i, :] = zrow
        return carry

    lax.fori_loop(0, GRP, _zfill, 0)

    def _zcopy(t, carry):
        pltpu.sync_copy(
            zero_v,
            acc_sh.at[pl.ds(s * RPT + t * GRP, GRP)],
        )
        return carry

    lax.fori_loop(0, RPT // GRP, _zcopy, 0)

    stage_src.wait()
    stage_dst.wait()
    stage_w.wait()
    stage_tab.wait()

    plsc.subcore_barrier()

    for b in range(NBUF):
        pltpu.async_copy(table_sh.at[src_v.at[b]], rows_v.at[b], gsems[b])

    def _outer(gg, carry):
        for b in range(NBUF):
            g = gg * NBUF + b
            pltpu.make_async_copy(
                table_sh.at[src_v.at[g]], rows_v.at[b], gsems[b]
            ).wait()

            def _scale(j, carry2, g=g, b=b):
                w16 = w_v[g, pl.ds(j * D_H, D_H)]
                for k in range(D_H):
                    rows_v[b, j * D_H + k, :] = rows_v[b, j * D_H + k, :] * w16[k]
                return carry2

            lax.fori_loop(0, GRP // D_H, _scale, 0)
            pltpu.sync_copy(rows_v.at[b], acc_sh.at[dst_v.at[g]], add=True)

            @pl.when(g + NBUF < GPW)
            def _(g=g, b=b):
                pltpu.async_copy(
                    table_sh.at[src_v.at[g + NBUF]], rows_v.at[b], gsems[b]
                )

        return carry

    lax.fori_loop(0, GPW // NBUF, _outer, 0)
    plsc.subcore_barrier()

    pltpu.sync_copy(
        acc_sh.at[pl.ds(s * RPT, RPT)],
        out_hbm.at[c, pl.ds(s * RPT, RPT)],
    )


_BN = 2000


def _p1_body(x_ref, wr_ref, wo_ref, xr_ref, xo_ref):
    x = x_ref[...]
    xr_ref[...] = jnp.dot(x, wr_ref[...], preferred_element_type=jnp.float32)
    xo_ref[...] = jnp.dot(x, wo_ref[...], preferred_element_type=jnp.float32)


def _p1(x, W_rel1, W_root1):
    return pl.pallas_call(
        _p1_body,
        grid=(N // _BN,),
        in_specs=[
            pl.BlockSpec((_BN, D_IN), lambda i: (i, 0)),
            pl.BlockSpec((D_IN, D_H), lambda i: (0, 0)),
            pl.BlockSpec((D_IN, D_H), lambda i: (0, 0)),
        ],
        out_specs=[
            pl.BlockSpec((_BN, D_H), lambda i: (i, 0)),
            pl.BlockSpec((_BN, D_H), lambda i: (i, 0)),
        ],
        out_shape=[
            jax.ShapeDtypeStruct((N_PAD, D_H), jnp.float32),
            jax.ShapeDtypeStruct((N_PAD, D_H), jnp.float32),
        ],
    )(x, W_rel1, W_root1)


def _p2_body(p0_ref, p1_ref, xo_ref, b_ref, h_ref):
    h = p0_ref[...] + p1_ref[...] + xo_ref[...] + b_ref[...]
    h_ref[...] = jnp.maximum(h, 0.0)


def _p2(p0, p1, xo, b1):
    return pl.pallas_call(
        _p2_body,
        grid=(N // _BN,),
        in_specs=[
            pl.BlockSpec((_BN, D_H), lambda i: (i, 0)),
            pl.BlockSpec((_BN, D_H), lambda i: (i, 0)),
            pl.BlockSpec((_BN, D_H), lambda i: (i, 0)),
            pl.BlockSpec((1, D_H), lambda i: (0, 0)),
        ],
        out_specs=pl.BlockSpec((_BN, D_H), lambda i: (i, 0)),
        out_shape=jax.ShapeDtypeStruct((N_PAD, D_H), jnp.float32),
    )(p0, p1, xo, b1)


def _p3_body(q0_ref, q1_ref, h_ref, wr_ref, wo_ref, b_ref, out_ref):
    agg = q0_ref[...] + q1_ref[...]
    out_ref[...] = (
        jnp.dot(agg, wr_ref[...], preferred_element_type=jnp.float32)
        + jnp.dot(h_ref[...], wo_ref[...], preferred_element_type=jnp.float32)
        + b_ref[...]
    )


def _p3(q0, q1, h, W_rel2, W_root2, b2):
    return pl.pallas_call(
        _p3_body,
        grid=(N // _BN,),
        in_specs=[
            pl.BlockSpec((_BN, D_H), lambda i: (i, 0)),
            pl.BlockSpec((_BN, D_H), lambda i: (i, 0)),
            pl.BlockSpec((_BN, D_H), lambda i: (i, 0)),
            pl.BlockSpec((D_H, D_OUT), lambda i: (0, 0)),
            pl.BlockSpec((D_H, D_OUT), lambda i: (0, 0)),
            pl.BlockSpec((1, D_OUT), lambda i: (0, 0)),
        ],
        out_specs=pl.BlockSpec((_BN, D_OUT), lambda i: (i, 0)),
        out_shape=jax.ShapeDtypeStruct((N, D_OUT), jnp.float32),
    )(q0, q1, h, W_rel2, W_root2, b2)


def kernel(x, edge_index, edge_attr, W_rel1, b_rel1, W_root1,
           W_rel2, b_rel2, W_root2):
    src = edge_index[0]
    dst = edge_index[1]

    pad = E_PAD - E
    src_p = jnp.concatenate([src, jnp.zeros((pad,), jnp.int32)]).reshape(NW * GPW, GRP)
    dst_p = jnp.concatenate([dst, jnp.zeros((pad,), jnp.int32)]).reshape(NW * GPW, GRP)
    w_p = jnp.concatenate([edge_attr, jnp.zeros((pad,), jnp.float32)]).reshape(NW * GPW, GRP)

    xr, xo = _p1(x, W_rel1, W_root1)
    p = _segsum_sc(xr, src_p, dst_p, w_p)
    h = _p2(p[0], p[1], xo, b_rel1.reshape(1, D_H))
    q = _segsum_sc(h, src_p, dst_p, w_p)
    return _p3(q[0], q[1], h, W_rel2, W_root2, b_rel2.reshape(1, D_OUT))


# speedup vs baseline: 27.1878x; 1.1744x over previous
import functools

import jax
import jax.numpy as jnp
from jax import lax
from jax.experimental import pallas as pl
from jax.experimental.pallas import tpu as pltpu
from jax.experimental.pallas import tpu_sc as plsc

N = 10000
E = 320000
D_IN = 128
D_H = 16
D_OUT = 128

NC = 2
NS = 16
NW = NC * NS
EPW = E // NW
GRP = 128
NG = EPW // GRP
TAIL = EPW - NG * GRP
NBUF = 2
N_PAD = 10240
RPT = N_PAD // NS

_mesh = plsc.VectorSubcoreMesh(
    core_axis_name="c", subcore_axis_name="s", num_cores=NC, num_subcores=NS
)

_SEG_SCRATCH = [
    pltpu.VMEM((EPW,), jnp.int32),
    pltpu.VMEM((EPW,), jnp.int32),
    pltpu.VMEM((EPW,), jnp.float32),
    pltpu.VMEM((NBUF, GRP, D_H), jnp.float32),
    pltpu.VMEM((TAIL, D_H), jnp.float32),
    pltpu.VMEM((GRP, D_H), jnp.float32),
    pltpu.VMEM_SHARED((N_PAD, D_H), jnp.float32),
    pltpu.VMEM_SHARED((N_PAD, D_H), jnp.float32),
    pltpu.SemaphoreType.DMA,
    pltpu.SemaphoreType.DMA,
    [pltpu.SemaphoreType.DMA] * NBUF,
]


def _stage_edges(src_hbm, dst_hbm, w_hbm, src_v, dst_v, w_v, wid, sem):
    eoff = wid * EPW
    return (
        pltpu.async_copy(src_hbm.at[pl.ds(eoff, EPW)], src_v, sem),
        pltpu.async_copy(dst_hbm.at[pl.ds(eoff, EPW)], dst_v, sem),
        pltpu.async_copy(w_hbm.at[pl.ds(eoff, EPW)], w_v, sem),
    )


def _zero_acc(zero_v, acc_sh, s):
    zrow = jnp.zeros((D_H,), jnp.float32)

    def _zfill(i, carry):
        zero_v[i, :] = zrow
        return carry

    lax.fori_loop(0, GRP, _zfill, 0)

    def _zcopy(t, carry):
        pltpu.sync_copy(zero_v, acc_sh.at[pl.ds(s * RPT + t * GRP, GRP)])
        return carry

    lax.fori_loop(0, RPT // GRP, _zcopy, 0)


def _edge_sweep(table_sh, acc_sh, src_v, dst_v, w_v, rows_v, tail_v, tsem, gsems):
    for b in range(NBUF):
        pltpu.async_copy(
            table_sh.at[src_v.at[pl.ds(b * GRP, GRP)]], rows_v.at[b], gsems[b]
        )
    tail_cp = pltpu.async_copy(
        table_sh.at[src_v.at[pl.ds(NG * GRP, TAIL)]], tail_v, tsem
    )

    def _outer(gg, carry):
        for b in range(NBUF):
            g = gg * NBUF + b
            goff = g * GRP
            pltpu.make_async_copy(
                table_sh.at[src_v.at[pl.ds(goff, GRP)]], rows_v.at[b], gsems[b]
            ).wait()

            def _scale(j, carry2, b=b, goff=goff):
                w16 = w_v[pl.ds(goff + j * D_H, D_H)]
                for k in range(D_H):
                    rows_v[b, j * D_H + k, :] = rows_v[b, j * D_H + k, :] * w16[k]
                return carry2

            lax.fori_loop(0, GRP // D_H, _scale, 0)
            pltpu.sync_copy(
                rows_v.at[b], acc_sh.at[dst_v.at[pl.ds(goff, GRP)]], add=True
            )

            @pl.when(g + NBUF < NG)
            def _(g=g, b=b):
                pltpu.async_copy(
                    table_sh.at[src_v.at[pl.ds((g + NBUF) * GRP, GRP)]],
                    rows_v.at[b],
                    gsems[b],
                )

        return carry

    lax.fori_loop(0, NG // NBUF, _outer, 0)

    tail_cp.wait()
    w16 = w_v[pl.ds(NG * GRP, D_H)]
    for k in range(TAIL):
        tail_v[k, :] = tail_v[k, :] * w16[k]
    pltpu.sync_copy(tail_v, acc_sh.at[dst_v.at[pl.ds(NG * GRP, TAIL)]], add=True)


@functools.partial(
    pl.kernel,
    mesh=_mesh,
    compiler_params=pltpu.CompilerParams(use_tc_tiling_on_sc=False),
    out_type=jax.ShapeDtypeStruct((NC, N_PAD, D_H), jnp.float32),
    scratch_types=_SEG_SCRATCH,
)
def _seg1_sc(table_hbm, src_hbm, dst_hbm, w_hbm, out_hbm,
             src_v, dst_v, w_v, rows_v, tail_v, zero_v, acc_sh, table_sh,
             stage_sem, tsem, gsems):
    c = lax.axis_index("c")
    s = lax.axis_index("s")
    wid = c * NS + s

    stages = _stage_edges(src_hbm, dst_hbm, w_hbm, src_v, dst_v, w_v, wid, stage_sem)
    tab_cp = pltpu.async_copy(
        table_hbm.at[pl.ds(s * RPT, RPT)], table_sh.at[pl.ds(s * RPT, RPT)], stage_sem
    )
    _zero_acc(zero_v, acc_sh, s)
    for st in stages:
        st.wait()
    tab_cp.wait()
    plsc.subcore_barrier()

    _edge_sweep(table_sh, acc_sh, src_v, dst_v, w_v, rows_v, tail_v, tsem, gsems)
    plsc.subcore_barrier()

    pltpu.sync_copy(
        acc_sh.at[pl.ds(s * RPT, RPT)], out_hbm.at[c, pl.ds(s * RPT, RPT)]
    )


@functools.partial(
    pl.kernel,
    mesh=_mesh,
    compiler_params=pltpu.CompilerParams(use_tc_tiling_on_sc=False),
    out_type=[
        jax.ShapeDtypeStruct((NC, N_PAD, D_H), jnp.float32),
        jax.ShapeDtypeStruct((N_PAD, D_H), jnp.float32),
    ],
    scratch_types=_SEG_SCRATCH + [
        pltpu.VMEM((RPT, D_H), jnp.float32),
        pltpu.VMEM((RPT, D_H), jnp.float32),
        pltpu.VMEM((RPT, D_H), jnp.float32),
        pltpu.VMEM((D_H,), jnp.float32),
    ],
)
def _seg2_sc(p_hbm, xo_hbm, b1_hbm, src_hbm, dst_hbm, w_hbm, out_hbm, h_hbm,
             src_v, dst_v, w_v, rows_v, tail_v, zero_v, acc_sh, table_sh,
             stage_sem, tsem, gsems, hb0, hb1, hb2, b1_v):
    c = lax.axis_index("c")
    s = lax.axis_index("s")
    wid = c * NS + s

    stages = _stage_edges(src_hbm, dst_hbm, w_hbm, src_v, dst_v, w_v, wid, stage_sem)
    noff = s * RPT
    st_p0 = pltpu.async_copy(p_hbm.at[0, pl.ds(noff, RPT)], hb0, stage_sem)
    st_p1 = pltpu.async_copy(p_hbm.at[1, pl.ds(noff, RPT)], hb1, stage_sem)
    st_xo = pltpu.async_copy(xo_hbm.at[pl.ds(noff, RPT)], hb2, stage_sem)
    st_b1 = pltpu.async_copy(b1_hbm, b1_v, stage_sem)
    _zero_acc(zero_v, acc_sh, s)
    for st in stages:
        st.wait()
    st_p0.wait()
    st_p1.wait()
    st_xo.wait()
    st_b1.wait()

    bvec = b1_v[...]

    def _hrow(i, carry):
        hrow = hb0[i, :] + hb1[i, :] + hb2[i, :] + bvec
        hb0[i, :] = jnp.maximum(hrow, 0.0)
        return carry

    lax.fori_loop(0, RPT, _hrow, 0)
    pltpu.sync_copy(hb0, table_sh.at[pl.ds(noff, RPT)])

    @pl.when(c == 0)
    def _():
        pltpu.sync_copy(hb0, h_hbm.at[pl.ds(noff, RPT)])

    plsc.subcore_barrier()

    _edge_sweep(table_sh, acc_sh, src_v, dst_v, w_v, rows_v, tail_v, tsem, gsems)
    plsc.subcore_barrier()

    pltpu.sync_copy(
        acc_sh.at[pl.ds(s * RPT, RPT)], out_hbm.at[c, pl.ds(s * RPT, RPT)]
    )


_BN = 2000


def _p1_body(x_ref, wr_ref, wo_ref, xr_ref, xo_ref):
    x = x_ref[...]
    xr_ref[...] = jnp.dot(x, wr_ref[...], preferred_element_type=jnp.float32)
    xo_ref[...] = jnp.dot(x, wo_ref[...], preferred_element_type=jnp.float32)


def _p1(x, W_rel1, W_root1):
    return pl.pallas_call(
        _p1_body,
        grid=(N // _BN,),
        in_specs=[
            pl.BlockSpec((_BN, D_IN), lambda i: (i, 0)),
            pl.BlockSpec((D_IN, D_H), lambda i: (0, 0)),
            pl.BlockSpec((D_IN, D_H), lambda i: (0, 0)),
        ],
        out_specs=[
            pl.BlockSpec((_BN, D_H), lambda i: (i, 0)),
            pl.BlockSpec((_BN, D_H), lambda i: (i, 0)),
        ],
        out_shape=[
            jax.ShapeDtypeStruct((N_PAD, D_H), jnp.float32),
            jax.ShapeDtypeStruct((N_PAD, D_H), jnp.float32),
        ],
    )(x, W_rel1, W_root1)


def _p3_body(q0_ref, q1_ref, h_ref, wr_ref, wo_ref, b_ref, out_ref):
    agg = q0_ref[...] + q1_ref[...]
    out_ref[...] = (
        jnp.dot(agg, wr_ref[...], preferred_element_type=jnp.float32)
        + jnp.dot(h_ref[...], wo_ref[...], preferred_element_type=jnp.float32)
        + b_ref[...]
    )


def _p3(q0, q1, h, W_rel2, W_root2, b2):
    return pl.pallas_call(
        _p3_body,
        grid=(N // _BN,),
        in_specs=[
            pl.BlockSpec((_BN, D_H), lambda i: (i, 0)),
            pl.BlockSpec((_BN, D_H), lambda i: (i, 0)),
            pl.BlockSpec((_BN, D_H), lambda i: (i, 0)),
            pl.BlockSpec((D_H, D_OUT), lambda i: (0, 0)),
            pl.BlockSpec((D_H, D_OUT), lambda i: (0, 0)),
            pl.BlockSpec((1, D_OUT), lambda i: (0, 0)),
        ],
        out_specs=pl.BlockSpec((_BN, D_OUT), lambda i: (i, 0)),
        out_shape=jax.ShapeDtypeStruct((N, D_OUT), jnp.float32),
    )(q0, q1, h, W_rel2, W_root2, b2)


def kernel(x, edge_index, edge_attr, W_rel1, b_rel1, W_root1,
           W_rel2, b_rel2, W_root2):
    src = edge_index[0]
    dst = edge_index[1]

    xr, xo = _p1(x, W_rel1, W_root1)
    p = _seg1_sc(xr, src, dst, edge_attr)
    q, h = _seg2_sc(p, xo, b_rel1, src, dst, edge_attr)
    return _p3(q[0], q[1], h, W_rel2, W_root2, b_rel2.reshape(1, D_OUT))


# speedup vs baseline: 29.0975x; 1.0702x over previous
import functools

import jax
import jax.numpy as jnp
from jax import lax
from jax.experimental import pallas as pl
from jax.experimental.pallas import tpu as pltpu
from jax.experimental.pallas import tpu_sc as plsc

N = 10000
E = 320000
D_IN = 128
D_H = 16
D_OUT = 128

NC = 2
NS = 16
NW = NC * NS
EPW = E // NW
GRP = 128
NG = EPW // GRP
TAIL = EPW - NG * GRP
NBUF = 2
N_PAD = 10240
RPT = N_PAD // NS

_mesh = plsc.VectorSubcoreMesh(
    core_axis_name="c", subcore_axis_name="s", num_cores=NC, num_subcores=NS
)

_SEG_SCRATCH = [
    pltpu.VMEM((EPW,), jnp.int32),
    pltpu.VMEM((EPW,), jnp.int32),
    pltpu.VMEM((EPW,), jnp.float32),
    pltpu.VMEM((NBUF, GRP, D_H), jnp.float32),
    pltpu.VMEM((TAIL, D_H), jnp.float32),
    pltpu.VMEM((GRP, D_H), jnp.float32),
    pltpu.VMEM_SHARED((N_PAD, D_H), jnp.float32),
    pltpu.VMEM_SHARED((N_PAD, D_H), jnp.float32),
    pltpu.SemaphoreType.DMA,
    pltpu.SemaphoreType.DMA,
    [pltpu.SemaphoreType.DMA] * NBUF,
]


def _stage_edges(ei_hbm, w_hbm, src_v, dst_v, w_v, wid, sem):
    eoff = wid * EPW
    return (
        pltpu.async_copy(ei_hbm.at[0, pl.ds(eoff, EPW)], src_v, sem),
        pltpu.async_copy(ei_hbm.at[1, pl.ds(eoff, EPW)], dst_v, sem),
        pltpu.async_copy(w_hbm.at[pl.ds(eoff, EPW)], w_v, sem),
    )


def _zero_acc(zero_v, acc_sh, s):
    zrow = jnp.zeros((D_H,), jnp.float32)

    def _zfill(i, carry):
        zero_v[i, :] = zrow
        return carry

    lax.fori_loop(0, GRP, _zfill, 0)

    def _zcopy(t, carry):
        pltpu.sync_copy(zero_v, acc_sh.at[pl.ds(s * RPT + t * GRP, GRP)])
        return carry

    lax.fori_loop(0, RPT // GRP, _zcopy, 0)


def _edge_sweep(table_sh, acc_sh, src_v, dst_v, w_v, rows_v, tail_v, tsem, gsems):
    for b in range(NBUF):
        pltpu.async_copy(
            table_sh.at[src_v.at[pl.ds(b * GRP, GRP)]], rows_v.at[b], gsems[b]
        )
    tail_cp = pltpu.async_copy(
        table_sh.at[src_v.at[pl.ds(NG * GRP, TAIL)]], tail_v, tsem
    )

    def _outer(gg, carry):
        for b in range(NBUF):
            g = gg * NBUF + b
            goff = g * GRP
            pltpu.make_async_copy(
                table_sh.at[src_v.at[pl.ds(goff, GRP)]], rows_v.at[b], gsems[b]
            ).wait()

            def _scale(j, carry2, b=b, goff=goff):
                w16 = w_v[pl.ds(goff + j * D_H, D_H)]
                for k in range(D_H):
                    rows_v[b, j * D_H + k, :] = rows_v[b, j * D_H + k, :] * w16[k]
                return carry2

            lax.fori_loop(0, GRP // D_H, _scale, 0)
            pltpu.sync_copy(
                rows_v.at[b], acc_sh.at[dst_v.at[pl.ds(goff, GRP)]], add=True
            )

            @pl.when(g + NBUF < NG)
            def _(g=g, b=b):
                pltpu.async_copy(
                    table_sh.at[src_v.at[pl.ds((g + NBUF) * GRP, GRP)]],
                    rows_v.at[b],
                    gsems[b],
                )

        return carry

    lax.fori_loop(0, NG // NBUF, _outer, 0)

    tail_cp.wait()
    w16 = w_v[pl.ds(NG * GRP, D_H)]
    for k in range(TAIL):
        tail_v[k, :] = tail_v[k, :] * w16[k]
    pltpu.sync_copy(tail_v, acc_sh.at[dst_v.at[pl.ds(NG * GRP, TAIL)]], add=True)


@functools.partial(
    pl.kernel,
    mesh=_mesh,
    compiler_params=pltpu.CompilerParams(use_tc_tiling_on_sc=False),
    out_type=jax.ShapeDtypeStruct((NC, N_PAD, D_H), jnp.float32),
    scratch_types=_SEG_SCRATCH,
)
def _seg1_sc(table_hbm, ei_hbm, w_hbm, out_hbm,
             src_v, dst_v, w_v, rows_v, tail_v, zero_v, acc_sh, table_sh,
             stage_sem, tsem, gsems):
    c = lax.axis_index("c")
    s = lax.axis_index("s")
    wid = c * NS + s

    stages = _stage_edges(ei_hbm, w_hbm, src_v, dst_v, w_v, wid, stage_sem)
    tab_cp = pltpu.async_copy(
        table_hbm.at[pl.ds(s * RPT, RPT)], table_sh.at[pl.ds(s * RPT, RPT)], stage_sem
    )
    _zero_acc(zero_v, acc_sh, s)
    for st in stages:
        st.wait()
    tab_cp.wait()
    plsc.subcore_barrier()

    _edge_sweep(table_sh, acc_sh, src_v, dst_v, w_v, rows_v, tail_v, tsem, gsems)
    plsc.subcore_barrier()

    pltpu.sync_copy(
        acc_sh.at[pl.ds(s * RPT, RPT)], out_hbm.at[c, pl.ds(s * RPT, RPT)]
    )


@functools.partial(
    pl.kernel,
    mesh=_mesh,
    compiler_params=pltpu.CompilerParams(use_tc_tiling_on_sc=False),
    out_type=[
        jax.ShapeDtypeStruct((NC, N_PAD, D_H), jnp.float32),
        jax.ShapeDtypeStruct((N_PAD, D_H), jnp.float32),
    ],
    scratch_types=_SEG_SCRATCH + [
        pltpu.VMEM((RPT, D_H), jnp.float32),
        pltpu.VMEM((RPT, D_H), jnp.float32),
        pltpu.VMEM((RPT, D_H), jnp.float32),
        pltpu.VMEM((D_H,), jnp.float32),
    ],
)
def _seg2_sc(p_hbm, xo_hbm, b1_hbm, ei_hbm, w_hbm, out_hbm, h_hbm,
             src_v, dst_v, w_v, rows_v, tail_v, zero_v, acc_sh, table_sh,
             stage_sem, tsem, gsems, hb0, hb1, hb2, b1_v):
    c = lax.axis_index("c")
    s = lax.axis_index("s")
    wid = c * NS + s

    stages = _stage_edges(ei_hbm, w_hbm, src_v, dst_v, w_v, wid, stage_sem)
    noff = s * RPT
    st_p0 = pltpu.async_copy(p_hbm.at[0, pl.ds(noff, RPT)], hb0, stage_sem)
    st_p1 = pltpu.async_copy(p_hbm.at[1, pl.ds(noff, RPT)], hb1, stage_sem)
    st_xo = pltpu.async_copy(xo_hbm.at[pl.ds(noff, RPT)], hb2, stage_sem)
    st_b1 = pltpu.async_copy(b1_hbm, b1_v, stage_sem)
    _zero_acc(zero_v, acc_sh, s)
    for st in stages:
        st.wait()
    st_p0.wait()
    st_p1.wait()
    st_xo.wait()
    st_b1.wait()

    bvec = b1_v[...]

    def _hrow(i, carry):
        hrow = hb0[i, :] + hb1[i, :] + hb2[i, :] + bvec
        hb0[i, :] = jnp.maximum(hrow, 0.0)
        return carry

    lax.fori_loop(0, RPT, _hrow, 0)
    pltpu.sync_copy(hb0, table_sh.at[pl.ds(noff, RPT)])

    @pl.when(c == 0)
    def _():
        pltpu.sync_copy(hb0, h_hbm.at[pl.ds(noff, RPT)])

    plsc.subcore_barrier()

    _edge_sweep(table_sh, acc_sh, src_v, dst_v, w_v, rows_v, tail_v, tsem, gsems)
    plsc.subcore_barrier()

    pltpu.sync_copy(
        acc_sh.at[pl.ds(s * RPT, RPT)], out_hbm.at[c, pl.ds(s * RPT, RPT)]
    )


_BN = 2000


def _p1_body(x_ref, wr_ref, wo_ref, xr_ref, xo_ref):
    x = x_ref[...]
    xr_ref[...] = jnp.dot(x, wr_ref[...], preferred_element_type=jnp.float32)
    xo_ref[...] = jnp.dot(x, wo_ref[...], preferred_element_type=jnp.float32)


def _p1(x, W_rel1, W_root1):
    return pl.pallas_call(
        _p1_body,
        grid=(N // _BN,),
        in_specs=[
            pl.BlockSpec((_BN, D_IN), lambda i: (i, 0)),
            pl.BlockSpec((D_IN, D_H), lambda i: (0, 0)),
            pl.BlockSpec((D_IN, D_H), lambda i: (0, 0)),
        ],
        out_specs=[
            pl.BlockSpec((_BN, D_H), lambda i: (i, 0)),
            pl.BlockSpec((_BN, D_H), lambda i: (i, 0)),
        ],
        out_shape=[
            jax.ShapeDtypeStruct((N_PAD, D_H), jnp.float32),
            jax.ShapeDtypeStruct((N_PAD, D_H), jnp.float32),
        ],
    )(x, W_rel1, W_root1)


def _p3_body(q0_ref, q1_ref, h_ref, wr_ref, wo_ref, b_ref, out_ref):
    agg = q0_ref[...] + q1_ref[...]
    out_ref[...] = (
        jnp.dot(agg, wr_ref[...], preferred_element_type=jnp.float32)
        + jnp.dot(h_ref[...], wo_ref[...], preferred_element_type=jnp.float32)
        + b_ref[...]
    )


def _p3(q0, q1, h, W_rel2, W_root2, b2):
    return pl.pallas_call(
        _p3_body,
        grid=(N // _BN,),
        in_specs=[
            pl.BlockSpec((_BN, D_H), lambda i: (i, 0)),
            pl.BlockSpec((_BN, D_H), lambda i: (i, 0)),
            pl.BlockSpec((_BN, D_H), lambda i: (i, 0)),
            pl.BlockSpec((D_H, D_OUT), lambda i: (0, 0)),
            pl.BlockSpec((D_H, D_OUT), lambda i: (0, 0)),
            pl.BlockSpec((1, D_OUT), lambda i: (0, 0)),
        ],
        out_specs=pl.BlockSpec((_BN, D_OUT), lambda i: (i, 0)),
        out_shape=jax.ShapeDtypeStruct((N, D_OUT), jnp.float32),
    )(q0, q1, h, W_rel2, W_root2, b2)


def kernel(x, edge_index, edge_attr, W_rel1, b_rel1, W_root1,
           W_rel2, b_rel2, W_root2):
    xr, xo = _p1(x, W_rel1, W_root1)
    p = _seg1_sc(xr, edge_index, edge_attr)
    q, h = _seg2_sc(p, xo, b_rel1, edge_index, edge_attr)
    return _p3(q[0], q[1], h, W_rel2, W_root2, b_rel2.reshape(1, D_OUT))


# speedup vs baseline: 31.4337x; 1.0803x over previous
import functools

import jax
import jax.numpy as jnp
from jax import lax
from jax.experimental import pallas as pl
from jax.experimental.pallas import tpu as pltpu
from jax.experimental.pallas import tpu_sc as plsc

N = 10000
E = 320000
D_IN = 128
D_H = 16
D_OUT = 128

NC = 2
NS = 16
NW = NC * NS
EPW = E // NW
GRP = 128
NG = EPW // GRP
TAIL = EPW - NG * GRP
NBUF = 4
N_PAD = 10240
RPT = N_PAD // NS

_mesh = plsc.VectorSubcoreMesh(
    core_axis_name="c", subcore_axis_name="s", num_cores=NC, num_subcores=NS
)

_SEG_SCRATCH = [
    pltpu.VMEM((EPW,), jnp.int32),
    pltpu.VMEM((EPW,), jnp.int32),
    pltpu.VMEM((EPW,), jnp.float32),
    pltpu.VMEM((NBUF, GRP, D_H), jnp.float32),
    pltpu.VMEM((TAIL, D_H), jnp.float32),
    pltpu.VMEM((GRP, D_H), jnp.float32),
    pltpu.VMEM_SHARED((N_PAD, D_H), jnp.float32),
    pltpu.VMEM_SHARED((N_PAD, D_H), jnp.float32),
    pltpu.SemaphoreType.DMA,
    pltpu.SemaphoreType.DMA,
    [pltpu.SemaphoreType.DMA] * NBUF,
    [pltpu.SemaphoreType.DMA] * NBUF,
]


def _stage_edges(ei_hbm, w_hbm, src_v, dst_v, w_v, wid, sem):
    eoff = wid * EPW
    return (
        pltpu.async_copy(ei_hbm.at[0, pl.ds(eoff, EPW)], src_v, sem),
        pltpu.async_copy(ei_hbm.at[1, pl.ds(eoff, EPW)], dst_v, sem),
        pltpu.async_copy(w_hbm.at[pl.ds(eoff, EPW)], w_v, sem),
    )


def _zero_acc(zero_v, acc_sh, s):
    zrow = jnp.zeros((D_H,), jnp.float32)

    def _zfill(i, carry):
        zero_v[i, :] = zrow
        return carry

    lax.fori_loop(0, GRP, _zfill, 0)

    def _zcopy(t, carry):
        pltpu.sync_copy(zero_v, acc_sh.at[pl.ds(s * RPT + t * GRP, GRP)])
        return carry

    lax.fori_loop(0, RPT // GRP, _zcopy, 0)


def _edge_sweep(table_sh, acc_sh, src_v, dst_v, w_v, rows_v, tail_v, tsem,
                gsems, ssems):
    def _scale(b, goff):
        def body(j, carry):
            w16 = w_v[pl.ds(goff + j * D_H, D_H)]
            for k in range(D_H):
                rows_v[b, j * D_H + k, :] = rows_v[b, j * D_H + k, :] * w16[k]
            return carry

        lax.fori_loop(0, GRP // D_H, body, 0)

    def _gather(g, b):
        pltpu.async_copy(
            table_sh.at[src_v.at[pl.ds(g * GRP, GRP)]], rows_v.at[b], gsems[b]
        )

    def _wait_gather(g, b):
        pltpu.make_async_copy(
            table_sh.at[src_v.at[pl.ds(g * GRP, GRP)]], rows_v.at[b], gsems[b]
        ).wait()

    def _scatter(g, b):
        pltpu.async_copy(
            rows_v.at[b], acc_sh.at[dst_v.at[pl.ds(g * GRP, GRP)]], ssems[b],
            add=True,
        )

    def _wait_scatter(g, b):
        pltpu.make_async_copy(
            rows_v.at[b], acc_sh.at[dst_v.at[pl.ds(g * GRP, GRP)]], ssems[b]
        ).wait()

    NMAIN = (NG - 2) // NBUF * NBUF

    for b in range(2):
        _gather(b, b)
    tail_cp = pltpu.async_copy(
        table_sh.at[src_v.at[pl.ds(NG * GRP, TAIL)]], tail_v, tsem
    )

    def _outer(gg, carry):
        g0 = gg * NBUF
        for u in range(NBUF):
            g = g0 + u
            b = u
            b2 = (u + 2) % NBUF
            _wait_gather(g, b)
            _scale(b, g * GRP)
            _scatter(g, b)

            @pl.when(g >= 2)
            def _(g=g, b2=b2):
                _wait_scatter(g - 2, b2)

            _gather(g + 2, b2)
        return carry

    lax.fori_loop(0, NMAIN // NBUF, _outer, 0)

    for g in range(NMAIN, NG):
        b = g % NBUF
        _wait_gather(g, b)
        _scale(b, g * GRP)
        _wait_scatter(g - NBUF + 2, (b + 2) % NBUF)
        _scatter(g, b)

    tail_cp.wait()
    w16 = w_v[pl.ds(NG * GRP, D_H)]
    for k in range(TAIL):
        tail_v[k, :] = tail_v[k, :] * w16[k]
    pltpu.sync_copy(tail_v, acc_sh.at[dst_v.at[pl.ds(NG * GRP, TAIL)]], add=True)

    for g in range(NG - NBUF + 2, NG):
        _wait_scatter(g, g % NBUF)


@functools.partial(
    pl.kernel,
    mesh=_mesh,
    compiler_params=pltpu.CompilerParams(use_tc_tiling_on_sc=False),
    out_type=jax.ShapeDtypeStruct((NC, N_PAD, D_H), jnp.float32),
    scratch_types=_SEG_SCRATCH,
)
def _seg1_sc(table_hbm, ei_hbm, w_hbm, out_hbm,
             src_v, dst_v, w_v, rows_v, tail_v, zero_v, acc_sh, table_sh,
             stage_sem, tsem, gsems, ssems):
    c = lax.axis_index("c")
    s = lax.axis_index("s")
    wid = c * NS + s

    stages = _stage_edges(ei_hbm, w_hbm, src_v, dst_v, w_v, wid, stage_sem)
    tab_cp = pltpu.async_copy(
        table_hbm.at[pl.ds(s * RPT, RPT)], table_sh.at[pl.ds(s * RPT, RPT)], stage_sem
    )
    _zero_acc(zero_v, acc_sh, s)
    for st in stages:
        st.wait()
    tab_cp.wait()
    plsc.subcore_barrier()

    _edge_sweep(table_sh, acc_sh, src_v, dst_v, w_v, rows_v, tail_v, tsem,
                gsems, ssems)
    plsc.subcore_barrier()

    pltpu.sync_copy(
        acc_sh.at[pl.ds(s * RPT, RPT)], out_hbm.at[c, pl.ds(s * RPT, RPT)]
    )


@functools.partial(
    pl.kernel,
    mesh=_mesh,
    compiler_params=pltpu.CompilerParams(use_tc_tiling_on_sc=False),
    out_type=[
        jax.ShapeDtypeStruct((NC, N_PAD, D_H), jnp.float32),
        jax.ShapeDtypeStruct((N_PAD, D_H), jnp.float32),
    ],
    scratch_types=_SEG_SCRATCH + [
        pltpu.VMEM((RPT, D_H), jnp.float32),
        pltpu.VMEM((RPT, D_H), jnp.float32),
        pltpu.VMEM((RPT, D_H), jnp.float32),
        pltpu.VMEM((D_H,), jnp.float32),
    ],
)
def _seg2_sc(p_hbm, xo_hbm, b1_hbm, ei_hbm, w_hbm, out_hbm, h_hbm,
             src_v, dst_v, w_v, rows_v, tail_v, zero_v, acc_sh, table_sh,
             stage_sem, tsem, gsems, ssems, hb0, hb1, hb2, b1_v):
    c = lax.axis_index("c")
    s = lax.axis_index("s")
    wid = c * NS + s

    stages = _stage_edges(ei_hbm, w_hbm, src_v, dst_v, w_v, wid, stage_sem)
    noff = s * RPT
    st_p0 = pltpu.async_copy(p_hbm.at[0, pl.ds(noff, RPT)], hb0, stage_sem)
    st_p1 = pltpu.async_copy(p_hbm.at[1, pl.ds(noff, RPT)], hb1, stage_sem)
    st_xo = pltpu.async_copy(xo_hbm.at[pl.ds(noff, RPT)], hb2, stage_sem)
    st_b1 = pltpu.async_copy(b1_hbm, b1_v, stage_sem)
    _zero_acc(zero_v, acc_sh, s)
    for st in stages:
        st.wait()
    st_p0.wait()
    st_p1.wait()
    st_xo.wait()
    st_b1.wait()

    bvec = b1_v[...]

    def _hrow(i, carry):
        hrow = hb0[i, :] + hb1[i, :] + hb2[i, :] + bvec
        hb0[i, :] = jnp.maximum(hrow, 0.0)
        return carry

    lax.fori_loop(0, RPT, _hrow, 0)
    pltpu.sync_copy(hb0, table_sh.at[pl.ds(noff, RPT)])

    @pl.when(c == 0)
    def _():
        pltpu.sync_copy(hb0, h_hbm.at[pl.ds(noff, RPT)])

    plsc.subcore_barrier()

    _edge_sweep(table_sh, acc_sh, src_v, dst_v, w_v, rows_v, tail_v, tsem,
                gsems, ssems)
    plsc.subcore_barrier()

    pltpu.sync_copy(
        acc_sh.at[pl.ds(s * RPT, RPT)], out_hbm.at[c, pl.ds(s * RPT, RPT)]
    )


_BN = 2000


def _p1_body(x_ref, wr_ref, wo_ref, xr_ref, xo_ref):
    x = x_ref[...]
    xr_ref[...] = jnp.dot(x, wr_ref[...], preferred_element_type=jnp.float32)
    xo_ref[...] = jnp.dot(x, wo_ref[...], preferred_element_type=jnp.float32)


def _p1(x, W_rel1, W_root1):
    return pl.pallas_call(
        _p1_body,
        grid=(N // _BN,),
        in_specs=[
            pl.BlockSpec((_BN, D_IN), lambda i: (i, 0)),
            pl.BlockSpec((D_IN, D_H), lambda i: (0, 0)),
            pl.BlockSpec((D_IN, D_H), lambda i: (0, 0)),
        ],
        out_specs=[
            pl.BlockSpec((_BN, D_H), lambda i: (i, 0)),
            pl.BlockSpec((_BN, D_H), lambda i: (i, 0)),
        ],
        out_shape=[
            jax.ShapeDtypeStruct((N_PAD, D_H), jnp.float32),
            jax.ShapeDtypeStruct((N_PAD, D_H), jnp.float32),
        ],
    )(x, W_rel1, W_root1)


def _p3_body(q0_ref, q1_ref, h_ref, wr_ref, wo_ref, b_ref, out_ref):
    agg = q0_ref[...] + q1_ref[...]
    out_ref[...] = (
        jnp.dot(agg, wr_ref[...], preferred_element_type=jnp.float32)
        + jnp.dot(h_ref[...], wo_ref[...], preferred_element_type=jnp.float32)
        + b_ref[...]
    )


def _p3(q0, q1, h, W_rel2, W_root2, b2):
    return pl.pallas_call(
        _p3_body,
        grid=(N // _BN,),
        in_specs=[
            pl.BlockSpec((_BN, D_H), lambda i: (i, 0)),
            pl.BlockSpec((_BN, D_H), lambda i: (i, 0)),
            pl.BlockSpec((_BN, D_H), lambda i: (i, 0)),
            pl.BlockSpec((D_H, D_OUT), lambda i: (0, 0)),
            pl.BlockSpec((D_H, D_OUT), lambda i: (0, 0)),
            pl.BlockSpec((1, D_OUT), lambda i: (0, 0)),
        ],
        out_specs=pl.BlockSpec((_BN, D_OUT), lambda i: (i, 0)),
        out_shape=jax.ShapeDtypeStruct((N, D_OUT), jnp.float32),
    )(q0, q1, h, W_rel2, W_root2, b2)


def kernel(x, edge_index, edge_attr, W_rel1, b_rel1, W_root1,
           W_rel2, b_rel2, W_root2):
    xr, xo = _p1(x, W_rel1, W_root1)
    p = _seg1_sc(xr, edge_index, edge_attr)
    q, h = _seg2_sc(p, xo, b_rel1, edge_index, edge_attr)
    return _p3(q[0], q[1], h, W_rel2, W_root2, b_rel2.reshape(1, D_OUT))


# speedup vs baseline: 36.4153x; 1.1585x over previous
import functools

import jax
import jax.numpy as jnp
from jax import lax
from jax.experimental import pallas as pl
from jax.experimental.pallas import tpu as pltpu
from jax.experimental.pallas import tpu_sc as plsc

N = 10000
E = 320000
D_IN = 128
D_H = 16
D_OUT = 128

NC = 2
NS = 16
NW = NC * NS
EPW = E // NW
GRP = 128
NG = EPW // GRP
TAIL = EPW - NG * GRP
NBUF = 4
N_PAD = 10240
RPT = N_PAD // NS

_mesh = plsc.VectorSubcoreMesh(
    core_axis_name="c", subcore_axis_name="s", num_cores=NC, num_subcores=NS
)

_SEG_SCRATCH = [
    pltpu.VMEM((EPW,), jnp.int32),
    pltpu.VMEM((EPW,), jnp.int32),
    pltpu.VMEM((EPW,), jnp.float32),
    pltpu.VMEM((NBUF, GRP, D_H), jnp.float32),
    pltpu.VMEM((TAIL, D_H), jnp.float32),
    pltpu.VMEM((GRP, D_H), jnp.float32),
    pltpu.VMEM_SHARED((N_PAD, D_H), jnp.float32),
    pltpu.VMEM_SHARED((N_PAD, D_H), jnp.float32),
    pltpu.SemaphoreType.DMA,
    pltpu.SemaphoreType.DMA,
    [pltpu.SemaphoreType.DMA] * NBUF,
    [pltpu.SemaphoreType.DMA] * NBUF,
]


def _stage_edges(ei_hbm, w_hbm, src_v, dst_v, w_v, wid, sem):
    eoff = wid * EPW
    return (
        pltpu.async_copy(ei_hbm.at[0, pl.ds(eoff, EPW)], src_v, sem),
        pltpu.async_copy(ei_hbm.at[1, pl.ds(eoff, EPW)], dst_v, sem),
        pltpu.async_copy(w_hbm.at[pl.ds(eoff, EPW)], w_v, sem),
    )


def _zero_acc(zero_v, acc_sh, s):
    zrow = jnp.zeros((D_H,), jnp.float32)

    def _zfill(i, carry):
        zero_v[i, :] = zrow
        return carry

    lax.fori_loop(0, GRP, _zfill, 0)

    def _zcopy(t, carry):
        pltpu.sync_copy(zero_v, acc_sh.at[pl.ds(s * RPT + t * GRP, GRP)])
        return carry

    lax.fori_loop(0, RPT // GRP, _zcopy, 0)


def _edge_sweep(table_sh, acc_sh, src_v, dst_v, w_v, rows_v, tail_v, tsem,
                gsems, ssems):
    def _scale(b, goff):
        def body(j, carry):
            w16 = w_v[pl.ds(goff + j * D_H, D_H)]
            for k in range(D_H):
                rows_v[b, j * D_H + k, :] = rows_v[b, j * D_H + k, :] * w16[k]
            return carry

        lax.fori_loop(0, GRP // D_H, body, 0)

    def _gather(g, b):
        pltpu.async_copy(
            table_sh.at[src_v.at[pl.ds(g * GRP, GRP)]], rows_v.at[b], gsems[b]
        )

    def _wait_gather(g, b):
        pltpu.make_async_copy(
            table_sh.at[src_v.at[pl.ds(g * GRP, GRP)]], rows_v.at[b], gsems[b]
        ).wait()

    def _scatter(g, b):
        pltpu.async_copy(
            rows_v.at[b], acc_sh.at[dst_v.at[pl.ds(g * GRP, GRP)]], ssems[b],
            add=True,
        )

    def _wait_scatter(g, b):
        pltpu.make_async_copy(
            rows_v.at[b], acc_sh.at[dst_v.at[pl.ds(g * GRP, GRP)]], ssems[b]
        ).wait()

    NMAIN = (NG - 2) // NBUF * NBUF

    for b in range(2):
        _gather(b, b)
    tail_cp = pltpu.async_copy(
        table_sh.at[src_v.at[pl.ds(NG * GRP, TAIL)]], tail_v, tsem
    )

    def _outer(gg, carry):
        g0 = gg * NBUF
        for u in range(NBUF):
            g = g0 + u
            b = u
            b2 = (u + 2) % NBUF
            _wait_gather(g, b)
            _scale(b, g * GRP)
            _scatter(g, b)

            @pl.when(g >= 2)
            def _(g=g, b2=b2):
                _wait_scatter(g - 2, b2)

            _gather(g + 2, b2)
        return carry

    lax.fori_loop(0, NMAIN // NBUF, _outer, 0)

    for g in range(NMAIN, NG):
        b = g % NBUF
        _wait_gather(g, b)
        _scale(b, g * GRP)
        _wait_scatter(g - NBUF + 2, (b + 2) % NBUF)
        _scatter(g, b)

    tail_cp.wait()
    w16 = w_v[pl.ds(NG * GRP, D_H)]
    for k in range(TAIL):
        tail_v[k, :] = tail_v[k, :] * w16[k]
    pltpu.sync_copy(tail_v, acc_sh.at[dst_v.at[pl.ds(NG * GRP, TAIL)]], add=True)

    for g in range(NG - NBUF + 2, NG):
        _wait_scatter(g, g % NBUF)


@functools.partial(
    pl.kernel,
    mesh=_mesh,
    compiler_params=pltpu.CompilerParams(use_tc_tiling_on_sc=False),
    out_type=jax.ShapeDtypeStruct((N_PAD, 128), jnp.float32),
    scratch_types=_SEG_SCRATCH,
)
def _seg1_sc(table_hbm, ei_hbm, w_hbm, out_hbm,
             src_v, dst_v, w_v, rows_v, tail_v, zero_v, acc_sh, table_sh,
             stage_sem, tsem, gsems, ssems):
    c = lax.axis_index("c")
    s = lax.axis_index("s")
    wid = c * NS + s

    stages = _stage_edges(ei_hbm, w_hbm, src_v, dst_v, w_v, wid, stage_sem)
    tab_cp = pltpu.async_copy(
        table_hbm.at[pl.ds(s * RPT, RPT), pl.ds(0, D_H)],
        table_sh.at[pl.ds(s * RPT, RPT)],
        stage_sem,
    )
    _zero_acc(zero_v, acc_sh, s)
    for st in stages:
        st.wait()
    tab_cp.wait()
    plsc.subcore_barrier()

    _edge_sweep(table_sh, acc_sh, src_v, dst_v, w_v, rows_v, tail_v, tsem,
                gsems, ssems)
    plsc.subcore_barrier()

    @pl.when(c == 0)
    def _():
        pltpu.sync_copy(
            acc_sh.at[pl.ds(s * RPT, RPT)],
            out_hbm.at[pl.ds(s * RPT, RPT), pl.ds(0, D_H)],
        )

    @pl.when(c == 1)
    def _():
        pltpu.sync_copy(
            acc_sh.at[pl.ds(s * RPT, RPT)],
            out_hbm.at[pl.ds(s * RPT, RPT), pl.ds(D_H, D_H)],
        )


@functools.partial(
    pl.kernel,
    mesh=_mesh,
    compiler_params=pltpu.CompilerParams(use_tc_tiling_on_sc=False),
    out_type=jax.ShapeDtypeStruct((N_PAD, 128), jnp.float32),
    scratch_types=_SEG_SCRATCH + [
        pltpu.VMEM((RPT, D_H), jnp.float32),
        pltpu.VMEM((RPT, D_H), jnp.float32),
        pltpu.VMEM((RPT, D_H), jnp.float32),
        pltpu.VMEM((D_H,), jnp.float32),
    ],
)
def _seg2_sc(p_hbm, xw_hbm, b1_hbm, ei_hbm, w_hbm, out_hbm,
             src_v, dst_v, w_v, rows_v, tail_v, zero_v, acc_sh, table_sh,
             stage_sem, tsem, gsems, ssems, hb0, hb1, hb2, b1_v):
    c = lax.axis_index("c")
    s = lax.axis_index("s")
    wid = c * NS + s

    stages = _stage_edges(ei_hbm, w_hbm, src_v, dst_v, w_v, wid, stage_sem)
    noff = s * RPT
    st_p0 = pltpu.async_copy(p_hbm.at[pl.ds(noff, RPT), pl.ds(0, D_H)], hb0, stage_sem)
    st_p1 = pltpu.async_copy(p_hbm.at[pl.ds(noff, RPT), pl.ds(D_H, D_H)], hb1, stage_sem)
    st_xo = pltpu.async_copy(xw_hbm.at[pl.ds(noff, RPT), pl.ds(D_H, D_H)], hb2, stage_sem)
    st_b1 = pltpu.async_copy(b1_hbm, b1_v, stage_sem)
    _zero_acc(zero_v, acc_sh, s)
    for st in stages:
        st.wait()
    st_p0.wait()
    st_p1.wait()
    st_xo.wait()
    st_b1.wait()

    bvec = b1_v[...]

    def _hrow(i, carry):
        hrow = hb0[i, :] + hb1[i, :] + hb2[i, :] + bvec
        hb0[i, :] = jnp.maximum(hrow, 0.0)
        return carry

    lax.fori_loop(0, RPT, _hrow, 0)
    pltpu.sync_copy(hb0, table_sh.at[pl.ds(noff, RPT)])

    @pl.when(c == 0)
    def _():
        pltpu.sync_copy(hb0, out_hbm.at[pl.ds(noff, RPT), pl.ds(2 * D_H, D_H)])

    plsc.subcore_barrier()

    _edge_sweep(table_sh, acc_sh, src_v, dst_v, w_v, rows_v, tail_v, tsem,
                gsems, ssems)
    plsc.subcore_barrier()

    @pl.when(c == 0)
    def _():
        pltpu.sync_copy(
            acc_sh.at[pl.ds(s * RPT, RPT)],
            out_hbm.at[pl.ds(s * RPT, RPT), pl.ds(0, D_H)],
        )

    @pl.when(c == 1)
    def _():
        pltpu.sync_copy(
            acc_sh.at[pl.ds(s * RPT, RPT)],
            out_hbm.at[pl.ds(s * RPT, RPT), pl.ds(D_H, D_H)],
        )


_BN = 2000


def _p1_body(x_ref, wr_ref, wo_ref, xw_ref):
    x = x_ref[...]
    xw_ref[:, 0:D_H] = jnp.dot(x, wr_ref[...], preferred_element_type=jnp.float32)
    xw_ref[:, D_H:2 * D_H] = jnp.dot(x, wo_ref[...], preferred_element_type=jnp.float32)


def _p1(x, W_rel1, W_root1):
    return pl.pallas_call(
        _p1_body,
        grid=(N // _BN,),
        in_specs=[
            pl.BlockSpec((_BN, D_IN), lambda i: (i, 0)),
            pl.BlockSpec((D_IN, D_H), lambda i: (0, 0)),
            pl.BlockSpec((D_IN, D_H), lambda i: (0, 0)),
        ],
        out_specs=pl.BlockSpec((_BN, 128), lambda i: (i, 0)),
        out_shape=jax.ShapeDtypeStruct((N_PAD, 128), jnp.float32),
    )(x, W_rel1, W_root1)


def _p3_body(qh_ref, wr_ref, wo_ref, b_ref, out_ref):
    v = qh_ref[...]
    agg = v[:, 0:D_H] + v[:, D_H:2 * D_H]
    h = v[:, 2 * D_H:3 * D_H]
    out_ref[...] = (
        jnp.dot(agg, wr_ref[...], preferred_element_type=jnp.float32)
        + jnp.dot(h, wo_ref[...], preferred_element_type=jnp.float32)
        + b_ref[...]
    )


def _p3(qh, W_rel2, W_root2, b2):
    return pl.pallas_call(
        _p3_body,
        grid=(N // _BN,),
        in_specs=[
            pl.BlockSpec((_BN, 128), lambda i: (i, 0)),
            pl.BlockSpec((D_H, D_OUT), lambda i: (0, 0)),
            pl.BlockSpec((D_H, D_OUT), lambda i: (0, 0)),
            pl.BlockSpec((1, D_OUT), lambda i: (0, 0)),
        ],
        out_specs=pl.BlockSpec((_BN, D_OUT), lambda i: (i, 0)),
        out_shape=jax.ShapeDtypeStruct((N, D_OUT), jnp.float32),
    )(qh, W_rel2, W_root2, b2)


def kernel(x, edge_index, edge_attr, W_rel1, b_rel1, W_root1,
           W_rel2, b_rel2, W_root2):
    xw = _p1(x, W_rel1, W_root1)
    p = _seg1_sc(xw, edge_index, edge_attr)
    qh = _seg2_sc(p, xw, b_rel1, edge_index, edge_attr)
    return _p3(qh, W_rel2, W_root2, b_rel2.reshape(1, D_OUT))


# speedup vs baseline: 36.5354x; 1.0033x over previous
import functools

import jax
import jax.numpy as jnp
from jax import lax
from jax.experimental import pallas as pl
from jax.experimental.pallas import tpu as pltpu
from jax.experimental.pallas import tpu_sc as plsc

N = 10000
E = 320000
D_IN = 128
D_H = 16
D_OUT = 128

NC = 2
NS = 16
NW = NC * NS
EPW = E // NW
GRP = 128
NG = EPW // GRP
TAIL = EPW - NG * GRP
NBUF = 4
N_PAD = 10240
RPT = N_PAD // NS

_mesh = plsc.VectorSubcoreMesh(
    core_axis_name="c", subcore_axis_name="s", num_cores=NC, num_subcores=NS
)

_SEG_SCRATCH = [
    pltpu.VMEM((EPW,), jnp.int32),
    pltpu.VMEM((EPW,), jnp.int32),
    pltpu.VMEM((EPW,), jnp.float32),
    pltpu.VMEM((NBUF, GRP, D_H), jnp.float32),
    pltpu.VMEM((TAIL, D_H), jnp.float32),
    pltpu.VMEM((GRP, D_H), jnp.float32),
    pltpu.VMEM_SHARED((N_PAD, D_H), jnp.float32),
    pltpu.VMEM_SHARED((N_PAD, D_H), jnp.float32),
    pltpu.SemaphoreType.DMA,
    pltpu.SemaphoreType.DMA,
    [pltpu.SemaphoreType.DMA] * NBUF,
    [pltpu.SemaphoreType.DMA] * NBUF,
]


def _stage_edges(ei_hbm, w_hbm, src_v, dst_v, w_v, wid, sem):
    eoff = wid * EPW
    return (
        pltpu.async_copy(ei_hbm.at[0, pl.ds(eoff, EPW)], src_v, sem),
        pltpu.async_copy(ei_hbm.at[1, pl.ds(eoff, EPW)], dst_v, sem),
        pltpu.async_copy(w_hbm.at[pl.ds(eoff, EPW)], w_v, sem),
    )


def _zero_acc(zero_v, acc_sh, s):
    zrow = jnp.zeros((D_H,), jnp.float32)

    def _zfill(i, carry):
        zero_v[i, :] = zrow
        return carry

    lax.fori_loop(0, GRP, _zfill, 0)

    def _zcopy(t, carry):
        pltpu.sync_copy(zero_v, acc_sh.at[pl.ds(s * RPT + t * GRP, GRP)])
        return carry

    lax.fori_loop(0, RPT // GRP, _zcopy, 0)


def _edge_sweep(table_sh, acc_sh, src_v, dst_v, w_v, rows_v, tail_v, tsem,
                gsems, ssems):
    def _scale(b, goff):
        def body(j, carry):
            w16 = w_v[pl.ds(goff + j * D_H, D_H)]
            for k in range(D_H):
                rows_v[b, j * D_H + k, :] = rows_v[b, j * D_H + k, :] * w16[k]
            return carry

        lax.fori_loop(0, GRP // D_H, body, 0)

    def _gather(g, b):
        pltpu.async_copy(
            table_sh.at[src_v.at[pl.ds(g * GRP, GRP)]], rows_v.at[b], gsems[b]
        )

    def _wait_gather(g, b):
        pltpu.make_async_copy(
            table_sh.at[src_v.at[pl.ds(g * GRP, GRP)]], rows_v.at[b], gsems[b]
        ).wait()

    def _scatter(g, b):
        pltpu.async_copy(
            rows_v.at[b], acc_sh.at[dst_v.at[pl.ds(g * GRP, GRP)]], ssems[b],
            add=True,
        )

    def _wait_scatter(g, b):
        pltpu.make_async_copy(
            rows_v.at[b], acc_sh.at[dst_v.at[pl.ds(g * GRP, GRP)]], ssems[b]
        ).wait()

    NMAIN = (NG - 2) // NBUF * NBUF

    for b in range(2):
        _gather(b, b)
    tail_cp = pltpu.async_copy(
        table_sh.at[src_v.at[pl.ds(NG * GRP, TAIL)]], tail_v, tsem
    )

    def _outer(gg, carry):
        g0 = gg * NBUF
        for u in range(NBUF):
            g = g0 + u
            b = u
            b2 = (u + 2) % NBUF
            _wait_gather(g, b)
            _scale(b, g * GRP)
            _scatter(g, b)

            @pl.when(g >= 2)
            def _(g=g, b2=b2):
                _wait_scatter(g - 2, b2)

            _gather(g + 2, b2)
        return carry

    lax.fori_loop(0, NMAIN // NBUF, _outer, 0)

    for g in range(NMAIN, NG):
        b = g % NBUF
        _wait_gather(g, b)
        _scale(b, g * GRP)
        _wait_scatter(g - NBUF + 2, (b + 2) % NBUF)
        _scatter(g, b)

    tail_cp.wait()
    w16 = w_v[pl.ds(NG * GRP, D_H)]
    for k in range(TAIL):
        tail_v[k, :] = tail_v[k, :] * w16[k]
    pltpu.sync_copy(tail_v, acc_sh.at[dst_v.at[pl.ds(NG * GRP, TAIL)]], add=True)

    for g in range(NG - NBUF + 2, NG):
        _wait_scatter(g, g % NBUF)


@functools.partial(
    pl.kernel,
    mesh=_mesh,
    compiler_params=pltpu.CompilerParams(use_tc_tiling_on_sc=False),
    out_type=jax.ShapeDtypeStruct((N_PAD, 128), jnp.float32),
    scratch_types=_SEG_SCRATCH,
)
def _seg1_sc(table_hbm, ei_hbm, w_hbm, out_hbm,
             src_v, dst_v, w_v, rows_v, tail_v, zero_v, acc_sh, table_sh,
             stage_sem, tsem, gsems, ssems):
    c = lax.axis_index("c")
    s = lax.axis_index("s")
    wid = c * NS + s

    stages = _stage_edges(ei_hbm, w_hbm, src_v, dst_v, w_v, wid, stage_sem)
    tab_cp = pltpu.async_copy(
        table_hbm.at[pl.ds(s * RPT, RPT), pl.ds(0, D_H)],
        table_sh.at[pl.ds(s * RPT, RPT)],
        stage_sem,
    )
    _zero_acc(zero_v, acc_sh, s)
    for st in stages:
        st.wait()
    tab_cp.wait()
    plsc.subcore_barrier()

    _edge_sweep(table_sh, acc_sh, src_v, dst_v, w_v, rows_v, tail_v, tsem,
                gsems, ssems)
    plsc.subcore_barrier()

    @pl.when(c == 0)
    def _():
        pltpu.sync_copy(
            acc_sh.at[pl.ds(s * RPT, RPT)],
            out_hbm.at[pl.ds(s * RPT, RPT), pl.ds(0, D_H)],
        )

    @pl.when(c == 1)
    def _():
        pltpu.sync_copy(
            acc_sh.at[pl.ds(s * RPT, RPT)],
            out_hbm.at[pl.ds(s * RPT, RPT), pl.ds(D_H, D_H)],
        )


@functools.partial(
    pl.kernel,
    mesh=_mesh,
    compiler_params=pltpu.CompilerParams(use_tc_tiling_on_sc=False),
    out_type=jax.ShapeDtypeStruct((N_PAD, 128), jnp.float32),
    scratch_types=_SEG_SCRATCH + [
        pltpu.VMEM((RPT, D_H), jnp.float32),
        pltpu.VMEM((RPT, D_H), jnp.float32),
        pltpu.VMEM((RPT, D_H), jnp.float32),
        pltpu.VMEM((D_H,), jnp.float32),
        pltpu.SemaphoreType.DMA,
    ],
)
def _seg2_sc(p_hbm, xw_hbm, b1_hbm, ei_hbm, w_hbm, out_hbm,
             src_v, dst_v, w_v, rows_v, tail_v, zero_v, acc_sh, table_sh,
             stage_sem, tsem, gsems, ssems, hb0, hb1, hb2, b1_v, hsem):
    c = lax.axis_index("c")
    s = lax.axis_index("s")
    wid = c * NS + s

    noff = s * RPT
    st_p0 = pltpu.async_copy(p_hbm.at[pl.ds(noff, RPT), pl.ds(0, D_H)], hb0, hsem)
    st_p1 = pltpu.async_copy(p_hbm.at[pl.ds(noff, RPT), pl.ds(D_H, D_H)], hb1, hsem)
    st_xo = pltpu.async_copy(xw_hbm.at[pl.ds(noff, RPT), pl.ds(D_H, D_H)], hb2, hsem)
    st_b1 = pltpu.async_copy(b1_hbm, b1_v, hsem)
    stages = _stage_edges(ei_hbm, w_hbm, src_v, dst_v, w_v, wid, stage_sem)
    _zero_acc(zero_v, acc_sh, s)
    st_p0.wait()
    st_p1.wait()
    st_xo.wait()
    st_b1.wait()

    bvec = b1_v[...]

    def _hrow(i, carry):
        hrow = hb0[i, :] + hb1[i, :] + hb2[i, :] + bvec
        hb0[i, :] = jnp.maximum(hrow, 0.0)
        return carry

    lax.fori_loop(0, RPT, _hrow, 0)
    pltpu.sync_copy(hb0, table_sh.at[pl.ds(noff, RPT)])

    @pl.when(c == 0)
    def _():
        pltpu.sync_copy(hb0, out_hbm.at[pl.ds(noff, RPT), pl.ds(2 * D_H, D_H)])

    for st in stages:
        st.wait()
    plsc.subcore_barrier()

    _edge_sweep(table_sh, acc_sh, src_v, dst_v, w_v, rows_v, tail_v, tsem,
                gsems, ssems)
    plsc.subcore_barrier()

    @pl.when(c == 0)
    def _():
        pltpu.sync_copy(
            acc_sh.at[pl.ds(s * RPT, RPT)],
            out_hbm.at[pl.ds(s * RPT, RPT), pl.ds(0, D_H)],
        )

    @pl.when(c == 1)
    def _():
        pltpu.sync_copy(
            acc_sh.at[pl.ds(s * RPT, RPT)],
            out_hbm.at[pl.ds(s * RPT, RPT), pl.ds(D_H, D_H)],
        )


_BN = 2000


def _p1_body(x_ref, wr_ref, wo_ref, xw_ref):
    x = x_ref[...]
    xw_ref[:, 0:D_H] = jnp.dot(x, wr_ref[...], preferred_element_type=jnp.float32)
    xw_ref[:, D_H:2 * D_H] = jnp.dot(x, wo_ref[...], preferred_element_type=jnp.float32)


def _p1(x, W_rel1, W_root1):
    return pl.pallas_call(
        _p1_body,
        grid=(N // _BN,),
        in_specs=[
            pl.BlockSpec((_BN, D_IN), lambda i: (i, 0)),
            pl.BlockSpec((D_IN, D_H), lambda i: (0, 0)),
            pl.BlockSpec((D_IN, D_H), lambda i: (0, 0)),
        ],
        out_specs=pl.BlockSpec((_BN, 128), lambda i: (i, 0)),
        out_shape=jax.ShapeDtypeStruct((N_PAD, 128), jnp.float32),
    )(x, W_rel1, W_root1)


def _p3_body(qh_ref, wr_ref, wo_ref, b_ref, out_ref):
    v = qh_ref[...]
    agg = v[:, 0:D_H] + v[:, D_H:2 * D_H]
    h = v[:, 2 * D_H:3 * D_H]
    out_ref[...] = (
        jnp.dot(agg, wr_ref[...], preferred_element_type=jnp.float32)
        + jnp.dot(h, wo_ref[...], preferred_element_type=jnp.float32)
        + b_ref[...]
    )


def _p3(qh, W_rel2, W_root2, b2):
    return pl.pallas_call(
        _p3_body,
        grid=(N // _BN,),
        in_specs=[
            pl.BlockSpec((_BN, 128), lambda i: (i, 0)),
            pl.BlockSpec((D_H, D_OUT), lambda i: (0, 0)),
            pl.BlockSpec((D_H, D_OUT), lambda i: (0, 0)),
            pl.BlockSpec((1, D_OUT), lambda i: (0, 0)),
        ],
        out_specs=pl.BlockSpec((_BN, D_OUT), lambda i: (i, 0)),
        out_shape=jax.ShapeDtypeStruct((N, D_OUT), jnp.float32),
    )(qh, W_rel2, W_root2, b2)


def kernel(x, edge_index, edge_attr, W_rel1, b_rel1, W_root1,
           W_rel2, b_rel2, W_root2):
    xw = _p1(x, W_rel1, W_root1)
    p = _seg1_sc(xw, edge_index, edge_attr)
    qh = _seg2_sc(p, xw, b_rel1, edge_index, edge_attr)
    return _p3(qh, W_rel2, W_root2, b_rel2.reshape(1, D_OUT))


# speedup vs baseline: 39.9370x; 1.0931x over previous
import functools

import jax
import jax.numpy as jnp
from jax import lax
from jax.experimental import pallas as pl
from jax.experimental.pallas import tpu as pltpu
from jax.experimental.pallas import tpu_sc as plsc

N = 10000
E = 320000
D_IN = 128
D_H = 16
D_OUT = 128

NC = 2
NS = 16
NW = NC * NS
EPW = E // NW
GRP = 128
NG = EPW // GRP
TAIL = EPW - NG * GRP
NBUF = 4
N_PAD = 10240
RPT = N_PAD // NS

_mesh = plsc.VectorSubcoreMesh(
    core_axis_name="c", subcore_axis_name="s", num_cores=NC, num_subcores=NS
)

_SEG_SCRATCH = [
    pltpu.VMEM((EPW,), jnp.int32),
    pltpu.VMEM((EPW,), jnp.int32),
    pltpu.VMEM((EPW,), jnp.float32),
    pltpu.VMEM((NBUF, GRP, D_H), jnp.float32),
    pltpu.VMEM((TAIL, D_H), jnp.float32),
    pltpu.VMEM((GRP, D_H), jnp.float32),
    pltpu.VMEM_SHARED((N_PAD, D_H), jnp.float32),
    pltpu.VMEM_SHARED((N_PAD, D_H), jnp.float32),
    pltpu.SemaphoreType.DMA,
    pltpu.SemaphoreType.DMA,
    [pltpu.SemaphoreType.DMA] * NBUF,
    [pltpu.SemaphoreType.DMA] * NBUF,
]


def _stage_edges(ei_hbm, w_hbm, src_v, dst_v, w_v, wid, sem):
    eoff = wid * EPW
    return (
        pltpu.async_copy(ei_hbm.at[0, pl.ds(eoff, EPW)], src_v, sem),
        pltpu.async_copy(ei_hbm.at[1, pl.ds(eoff, EPW)], dst_v, sem),
        pltpu.async_copy(w_hbm.at[pl.ds(eoff, EPW)], w_v, sem),
    )


def _zero_acc(zero_v, acc_sh, s):
    zrow = jnp.zeros((D_H,), jnp.float32)

    def _zfill(i, carry):
        zero_v[i, :] = zrow
        return carry

    lax.fori_loop(0, GRP, _zfill, 0)

    def _zcopy(t, carry):
        pltpu.sync_copy(zero_v, acc_sh.at[pl.ds(s * RPT + t * GRP, GRP)])
        return carry

    lax.fori_loop(0, RPT // GRP, _zcopy, 0)


def _edge_sweep(table_sh, acc_sh, src_v, dst_v, w_v, rows_v, tail_v, tsem,
                gsems, ssems):
    def _scale(b, goff):
        def body(j, carry):
            w16 = w_v[pl.ds(goff + j * D_H, D_H)]
            for k in range(D_H):
                rows_v[b, j * D_H + k, :] = rows_v[b, j * D_H + k, :] * w16[k]
            return carry

        lax.fori_loop(0, GRP // D_H, body, 0)

    def _gather(g, b):
        pltpu.async_copy(
            table_sh.at[src_v.at[pl.ds(g * GRP, GRP)]], rows_v.at[b], gsems[b]
        )

    def _wait_gather(g, b):
        pltpu.make_async_copy(
            table_sh.at[src_v.at[pl.ds(g * GRP, GRP)]], rows_v.at[b], gsems[b]
        ).wait()

    def _scatter(g, b):
        pltpu.async_copy(
            rows_v.at[b], acc_sh.at[dst_v.at[pl.ds(g * GRP, GRP)]], ssems[b],
            add=True,
        )

    def _wait_scatter(g, b):
        pltpu.make_async_copy(
            rows_v.at[b], acc_sh.at[dst_v.at[pl.ds(g * GRP, GRP)]], ssems[b]
        ).wait()

    NMAIN = (NG - 2) // NBUF * NBUF

    for b in range(2):
        _gather(b, b)
    tail_cp = pltpu.async_copy(
        table_sh.at[src_v.at[pl.ds(NG * GRP, TAIL)]], tail_v, tsem
    )

    def _outer(gg, carry):
        g0 = gg * NBUF
        for u in range(NBUF):
            g = g0 + u
            b = u
            b2 = (u + 2) % NBUF
            _wait_gather(g, b)
            _scale(b, g * GRP)
            _scatter(g, b)

            @pl.when(g >= 2)
            def _(g=g, b2=b2):
                _wait_scatter(g - 2, b2)

            _gather(g + 2, b2)
        return carry

    lax.fori_loop(0, NMAIN // NBUF, _outer, 0)

    for g in range(NMAIN, NG):
        b = g % NBUF
        _wait_gather(g, b)
        _scale(b, g * GRP)
        _wait_scatter(g - NBUF + 2, (b + 2) % NBUF)
        _scatter(g, b)

    tail_cp.wait()
    w16 = w_v[pl.ds(NG * GRP, D_H)]
    for k in range(TAIL):
        tail_v[k, :] = tail_v[k, :] * w16[k]
    pltpu.sync_copy(tail_v, acc_sh.at[dst_v.at[pl.ds(NG * GRP, TAIL)]], add=True)

    for g in range(NG - NBUF + 2, NG):
        _wait_scatter(g, g % NBUF)


@functools.partial(
    pl.kernel,
    mesh=_mesh,
    compiler_params=pltpu.CompilerParams(use_tc_tiling_on_sc=False),
    out_type=[
        jax.ShapeDtypeStruct((N_PAD, 128), jnp.float32),
        jax.ShapeDtypeStruct((N_PAD, 32), jnp.float32),
    ],
    scratch_types=_SEG_SCRATCH + [
        pltpu.VMEM((RPT, D_H), jnp.float32),
        pltpu.VMEM((RPT, D_H), jnp.float32),
        pltpu.VMEM((RPT, D_H), jnp.float32),
        pltpu.VMEM((D_H,), jnp.float32),
        pltpu.SemaphoreType.DMA,
        pltpu.SemaphoreType.REGULAR,
    ],
)
def _gcn_sc(xw_hbm, b1_hbm, ei_hbm, w_hbm, out_hbm, p_hbm,
            src_v, dst_v, w_v, rows_v, tail_v, zero_v, acc_sh, table_sh,
            stage_sem, tsem, gsems, ssems, hb0, hb1, hb2, b1_v, hsem, xsem):
    c = lax.axis_index("c")
    s = lax.axis_index("s")
    wid = c * NS + s
    noff = s * RPT

    st_xo = pltpu.async_copy(
        xw_hbm.at[pl.ds(noff, RPT), pl.ds(D_H, D_H)], hb2, hsem
    )
    st_b1 = pltpu.async_copy(b1_hbm, b1_v, hsem)
    stages = _stage_edges(ei_hbm, w_hbm, src_v, dst_v, w_v, wid, stage_sem)
    tab_cp = pltpu.async_copy(
        xw_hbm.at[pl.ds(noff, RPT), pl.ds(0, D_H)],
        table_sh.at[pl.ds(noff, RPT)],
        stage_sem,
    )
    _zero_acc(zero_v, acc_sh, s)
    for st in stages:
        st.wait()
    tab_cp.wait()
    plsc.subcore_barrier()

    _edge_sweep(table_sh, acc_sh, src_v, dst_v, w_v, rows_v, tail_v, tsem,
                gsems, ssems)
    plsc.subcore_barrier()

    pltpu.sync_copy(acc_sh.at[pl.ds(noff, RPT)], hb0)

    @pl.when(c == 0)
    def _():
        pltpu.sync_copy(hb0, p_hbm.at[pl.ds(noff, RPT), pl.ds(0, D_H)])

    @pl.when(c == 1)
    def _():
        pltpu.sync_copy(hb0, p_hbm.at[pl.ds(noff, RPT), pl.ds(D_H, D_H)])

    _zero_acc(zero_v, acc_sh, s)
    plsc.subcore_barrier()

    @pl.when(s == 0)
    def _():
        pltpu.semaphore_signal(xsem, 1, core_index=1 - c)
        pl.semaphore_wait(xsem, 1)

    plsc.subcore_barrier()

    @pl.when(c == 0)
    def _():
        pltpu.sync_copy(p_hbm.at[pl.ds(noff, RPT), pl.ds(D_H, D_H)], hb1)

    @pl.when(c == 1)
    def _():
        pltpu.sync_copy(p_hbm.at[pl.ds(noff, RPT), pl.ds(0, D_H)], hb1)

    st_xo.wait()
    st_b1.wait()

    bvec = b1_v[...]

    def _hrow(i, carry):
        hrow = hb0[i, :] + hb1[i, :] + hb2[i, :] + bvec
        hb0[i, :] = jnp.maximum(hrow, 0.0)
        return carry

    lax.fori_loop(0, RPT, _hrow, 0)
    pltpu.sync_copy(hb0, table_sh.at[pl.ds(noff, RPT)])

    @pl.when(c == 0)
    def _():
        pltpu.sync_copy(hb0, out_hbm.at[pl.ds(noff, RPT), pl.ds(2 * D_H, D_H)])

    plsc.subcore_barrier()

    _edge_sweep(table_sh, acc_sh, src_v, dst_v, w_v, rows_v, tail_v, tsem,
                gsems, ssems)
    plsc.subcore_barrier()

    @pl.when(c == 0)
    def _():
        pltpu.sync_copy(
            acc_sh.at[pl.ds(s * RPT, RPT)],
            out_hbm.at[pl.ds(s * RPT, RPT), pl.ds(0, D_H)],
        )

    @pl.when(c == 1)
    def _():
        pltpu.sync_copy(
            acc_sh.at[pl.ds(s * RPT, RPT)],
            out_hbm.at[pl.ds(s * RPT, RPT), pl.ds(D_H, D_H)],
        )


_BN = 2000


def _p1_body(x_ref, wr_ref, wo_ref, xw_ref):
    x = x_ref[...]
    xw_ref[:, 0:D_H] = jnp.dot(x, wr_ref[...], preferred_element_type=jnp.float32)
    xw_ref[:, D_H:2 * D_H] = jnp.dot(x, wo_ref[...], preferred_element_type=jnp.float32)


def _p1(x, W_rel1, W_root1):
    return pl.pallas_call(
        _p1_body,
        grid=(N // _BN,),
        in_specs=[
            pl.BlockSpec((_BN, D_IN), lambda i: (i, 0)),
            pl.BlockSpec((D_IN, D_H), lambda i: (0, 0)),
            pl.BlockSpec((D_IN, D_H), lambda i: (0, 0)),
        ],
        out_specs=pl.BlockSpec((_BN, 128), lambda i: (i, 0)),
        out_shape=jax.ShapeDtypeStruct((N_PAD, 128), jnp.float32),
    )(x, W_rel1, W_root1)


def _p3_body(qh_ref, wr_ref, wo_ref, b_ref, out_ref):
    v = qh_ref[...]
    agg = v[:, 0:D_H] + v[:, D_H:2 * D_H]
    h = v[:, 2 * D_H:3 * D_H]
    out_ref[...] = (
        jnp.dot(agg, wr_ref[...], preferred_element_type=jnp.float32)
        + jnp.dot(h, wo_ref[...], preferred_element_type=jnp.float32)
        + b_ref[...]
    )


def _p3(qh, W_rel2, W_root2, b2):
    return pl.pallas_call(
        _p3_body,
        grid=(N // _BN,),
        in_specs=[
            pl.BlockSpec((_BN, 128), lambda i: (i, 0)),
            pl.BlockSpec((D_H, D_OUT), lambda i: (0, 0)),
            pl.BlockSpec((D_H, D_OUT), lambda i: (0, 0)),
            pl.BlockSpec((1, D_OUT), lambda i: (0, 0)),
        ],
        out_specs=pl.BlockSpec((_BN, D_OUT), lambda i: (i, 0)),
        out_shape=jax.ShapeDtypeStruct((N, D_OUT), jnp.float32),
    )(qh, W_rel2, W_root2, b2)


def kernel(x, edge_index, edge_attr, W_rel1, b_rel1, W_root1,
           W_rel2, b_rel2, W_root2):
    xw = _p1(x, W_rel1, W_root1)
    qh, _ = _gcn_sc(xw, b_rel1, edge_index, edge_attr)
    return _p3(qh, W_rel2, W_root2, b_rel2.reshape(1, D_OUT))
